# Initial kernel scaffold; baseline (speedup 1.0000x reference)
import jax
import jax.numpy as jnp
from jax.experimental import pallas as pl


def kernel(points3d_obs, points3d_pred):
    raise NotImplementedError("write your pallas kernel here")


# speedup vs baseline: 2.1819x; 2.1819x over previous
import functools

import jax
import jax.numpy as jnp
from jax.experimental import pallas as pl


def _chamfer_body(pred_ref, obs_ref, out_ref, *, n_pred, n_obs, chunk):
    p = pred_ref[0]
    ot = obs_ref[0]
    p_norm = jnp.sum(p * p, axis=1, keepdims=True)
    o_norm = jnp.sum(ot * ot, axis=0, keepdims=True)
    lhs = jnp.concatenate([p, p_norm], axis=1)
    rhs = jnp.concatenate(
        [-2.0 * ot, jnp.ones((1, n_obs), jnp.float32)], axis=0)
    acc = jnp.full((1, n_obs), jnp.inf, dtype=jnp.float32)
    for j0 in range(0, n_pred, chunk):
        m = jax.lax.dot_general(
            lhs[j0:j0 + chunk, :], rhs,
            dimension_numbers=(((1,), (0,)), ((), ())),
            precision=jax.lax.Precision.HIGHEST,
            preferred_element_type=jnp.float32)
        acc = jnp.minimum(acc, jnp.min(m, axis=0, keepdims=True))
    d2 = jnp.maximum(acc + o_norm, 0.0)
    out_ref[0] = jnp.sqrt(d2)


def _median(x, n):
    k1 = n // 2
    k2 = k1 + 1
    lo0 = jnp.min(x) - 1.0
    hi0 = jnp.max(x)

    def body(_, carry):
        lo1, hi1, lo2, hi2 = carry
        m1 = 0.5 * (lo1 + hi1)
        m2 = 0.5 * (lo2 + hi2)
        c1 = jnp.sum(jnp.where(x <= m1, 1.0, 0.0))
        c2 = jnp.sum(jnp.where(x <= m2, 1.0, 0.0))
        t1 = c1 >= k1
        t2 = c2 >= k2
        return (jnp.where(t1, lo1, m1), jnp.where(t1, m1, hi1),
                jnp.where(t2, lo2, m2), jnp.where(t2, m2, hi2))

    lo1, hi1, lo2, hi2 = jax.lax.fori_loop(0, 48, body, (lo0, hi0, lo0, hi0))
    return 0.5 * (hi1 + hi2)


def _loss_body(res_ref, out_ref, *, n_batch, n_row):
    total = jnp.float32(0.0)
    for b in range(n_batch):
        x = res_ref[b]
        med = _median(x, n_row)
        mad = _median(jnp.abs(x - med), n_row)
        denom = (mad / 0.67449) * 4.6851
        nr = x / denom
        w = jnp.where(nr >= 1.0, 0.0, (1.0 - nr * nr) ** 2)
        total = total + jnp.sum(w * x * x)
    out_ref[...] = jnp.broadcast_to(0.5 * total, (1, 1))


def kernel(points3d_obs, points3d_pred):
    B, T, n_obs, _ = points3d_obs.shape
    n_pred = points3d_pred.shape[2]
    bt = B * T
    pred = points3d_pred.reshape(bt, n_pred, 3)
    obs_t = points3d_obs.reshape(bt, n_obs, 3).transpose(0, 2, 1)

    res = pl.pallas_call(
        functools.partial(_chamfer_body, n_pred=n_pred, n_obs=n_obs, chunk=512),
        grid=(bt,),
        in_specs=[
            pl.BlockSpec((1, n_pred, 3), lambda f: (f, 0, 0)),
            pl.BlockSpec((1, 3, n_obs), lambda f: (f, 0, 0)),
        ],
        out_specs=pl.BlockSpec((1, 1, n_obs), lambda f: (f, 0, 0)),
        out_shape=jax.ShapeDtypeStruct((bt, 1, n_obs), jnp.float32),
    )(pred, obs_t)

    n_row = T * n_obs
    res3 = res.reshape(B, n_row // 128, 128)
    loss = pl.pallas_call(
        functools.partial(_loss_body, n_batch=B, n_row=n_row),
        in_specs=[pl.BlockSpec((B, n_row // 128, 128), lambda: (0, 0, 0))],
        out_specs=pl.BlockSpec((1, 1), lambda: (0, 0)),
        out_shape=jax.ShapeDtypeStruct((1, 1), jnp.float32),
    )(res3)
    return loss[0, 0]


# speedup vs baseline: 6.4570x; 2.9593x over previous
import functools

import jax
import jax.numpy as jnp
from jax.experimental import pallas as pl


def _chamfer_body(pred_ref, obs_ref, out_ref, *, n_pred, n_obs, chunk):
    p = pred_ref[0]
    ot = obs_ref[0]
    p_norm = jnp.sum(p * p, axis=1, keepdims=True)
    o_norm = jnp.sum(ot * ot, axis=0, keepdims=True)
    lhs = jnp.concatenate([p, p_norm], axis=1)
    rhs = jnp.concatenate(
        [-2.0 * ot, jnp.ones((1, n_obs), jnp.float32)], axis=0)
    lhs_hi = lhs.astype(jnp.bfloat16)
    lhs_lo = (lhs - lhs_hi.astype(jnp.float32)).astype(jnp.bfloat16)
    rhs_hi = rhs.astype(jnp.bfloat16)
    rhs_lo = (rhs - rhs_hi.astype(jnp.float32)).astype(jnp.bfloat16)
    lhs_cat = jnp.concatenate([lhs_hi, lhs_hi, lhs_lo, lhs_lo], axis=1)
    rhs_cat = jnp.concatenate([rhs_hi, rhs_lo, rhs_hi, rhs_lo], axis=0)
    acc = jnp.full((1, n_obs), jnp.inf, dtype=jnp.float32)
    for j0 in range(0, n_pred, chunk):
        m = jax.lax.dot_general(
            lhs_cat[j0:j0 + chunk, :], rhs_cat,
            dimension_numbers=(((1,), (0,)), ((), ())),
            preferred_element_type=jnp.float32)
        acc = jnp.minimum(acc, jnp.min(m, axis=0, keepdims=True))
    d2 = jnp.maximum(acc + o_norm, 0.0)
    out_ref[0] = jnp.sqrt(d2)


def _median_pair(xs, n):
    k = n // 2
    bounds = []
    for x in xs:
        bounds.extend([jnp.min(x) - 1.0, jnp.max(x)])

    def body(_, carry):
        out = []
        for i, x in enumerate(xs):
            lo, hi = carry[2 * i], carry[2 * i + 1]
            m = 0.5 * (lo + hi)
            c = jnp.sum(jnp.where(x <= m, 1.0, 0.0))
            t = c >= k
            out.extend([jnp.where(t, lo, m), jnp.where(t, m, hi)])
        return tuple(out)

    carry = jax.lax.fori_loop(0, 40, body, tuple(bounds))
    meds = []
    for i, x in enumerate(xs):
        v1 = carry[2 * i + 1]
        succ = jnp.min(jnp.where(x > v1, x, jnp.inf))
        v2 = jnp.where(jnp.sum(jnp.where(x <= v1, 1.0, 0.0)) >= k + 1, v1, succ)
        meds.append(0.5 * (v1 + v2))
    return meds


def _loss_body(res_ref, out_ref, *, n_batch, n_row):
    rows = [res_ref[b] for b in range(n_batch)]
    meds = _median_pair(rows, n_row)
    devs = [jnp.abs(x - m) for x, m in zip(rows, meds)]
    mads = _median_pair(devs, n_row)
    total = jnp.float32(0.0)
    for x, mad in zip(rows, mads):
        denom = (mad / 0.67449) * 4.6851
        nr = x / denom
        w = jnp.where(nr >= 1.0, 0.0, (1.0 - nr * nr) ** 2)
        total = total + jnp.sum(w * x * x)
    out_ref[...] = jnp.broadcast_to(0.5 * total, (1, 1))


def kernel(points3d_obs, points3d_pred):
    B, T, n_obs, _ = points3d_obs.shape
    n_pred = points3d_pred.shape[2]
    bt = B * T
    pred = points3d_pred.reshape(bt, n_pred, 3)
    obs_t = points3d_obs.reshape(bt, n_obs, 3).transpose(0, 2, 1)

    res = pl.pallas_call(
        functools.partial(_chamfer_body, n_pred=n_pred, n_obs=n_obs, chunk=512),
        grid=(bt,),
        in_specs=[
            pl.BlockSpec((1, n_pred, 3), lambda f: (f, 0, 0)),
            pl.BlockSpec((1, 3, n_obs), lambda f: (f, 0, 0)),
        ],
        out_specs=pl.BlockSpec((1, 1, n_obs), lambda f: (f, 0, 0)),
        out_shape=jax.ShapeDtypeStruct((bt, 1, n_obs), jnp.float32),
    )(pred, obs_t)

    n_row = T * n_obs
    res3 = res.reshape(B, n_row // 128, 128)
    loss = pl.pallas_call(
        functools.partial(_loss_body, n_batch=B, n_row=n_row),
        in_specs=[pl.BlockSpec((B, n_row // 128, 128), lambda: (0, 0, 0))],
        out_specs=pl.BlockSpec((1, 1), lambda: (0, 0)),
        out_shape=jax.ShapeDtypeStruct((1, 1), jnp.float32),
    )(res3)
    return loss[0, 0]


# speedup vs baseline: 6.7802x; 1.0501x over previous
import functools

import jax
import jax.numpy as jnp
from jax.experimental import pallas as pl


def _chamfer_body(lhs_ref, rhs_ref, onorm_ref, out_ref, *, n_pred, n_obs,
                  chunk):
    lhs_cat = lhs_ref[0]
    rhs_cat = rhs_ref[0]
    acc = jnp.full((1, n_obs), jnp.inf, dtype=jnp.float32)
    for j0 in range(0, n_pred, chunk):
        m = jax.lax.dot_general(
            lhs_cat[j0:j0 + chunk, :], rhs_cat,
            dimension_numbers=(((1,), (0,)), ((), ())),
            preferred_element_type=jnp.float32)
        acc = jnp.minimum(acc, jnp.min(m, axis=0, keepdims=True))
    d2 = jnp.maximum(acc + onorm_ref[0], 0.0)
    out_ref[0] = jnp.sqrt(d2)


def _median_pair(xs, n):
    k = n // 2
    bounds = []
    for x in xs:
        bounds.extend([jnp.min(x) - 1.0, jnp.max(x)])

    def body(_, carry):
        out = []
        for i, x in enumerate(xs):
            lo, hi = carry[2 * i], carry[2 * i + 1]
            m = 0.5 * (lo + hi)
            c = jnp.sum(jnp.where(x <= m, 1.0, 0.0))
            t = c >= k
            out.extend([jnp.where(t, lo, m), jnp.where(t, m, hi)])
        return tuple(out)

    carry = jax.lax.fori_loop(0, 40, body, tuple(bounds))
    meds = []
    for i, x in enumerate(xs):
        v1 = carry[2 * i + 1]
        succ = jnp.min(jnp.where(x > v1, x, jnp.inf))
        v2 = jnp.where(jnp.sum(jnp.where(x <= v1, 1.0, 0.0)) >= k + 1, v1, succ)
        meds.append(0.5 * (v1 + v2))
    return meds


def _loss_body(res_ref, out_ref, *, n_batch, n_row):
    rows = [res_ref[b] for b in range(n_batch)]
    meds = _median_pair(rows, n_row)
    devs = [jnp.abs(x - m) for x, m in zip(rows, meds)]
    mads = _median_pair(devs, n_row)
    total = jnp.float32(0.0)
    for x, mad in zip(rows, mads):
        denom = (mad / 0.67449) * 4.6851
        nr = x / denom
        w = jnp.where(nr >= 1.0, 0.0, (1.0 - nr * nr) ** 2)
        total = total + jnp.sum(w * x * x)
    out_ref[...] = jnp.broadcast_to(0.5 * total, (1, 1))


def _split_bf16(x):
    hi = x.astype(jnp.bfloat16)
    lo = (x - hi.astype(jnp.float32)).astype(jnp.bfloat16)
    return hi, lo


def kernel(points3d_obs, points3d_pred):
    B, T, n_obs, _ = points3d_obs.shape
    n_pred = points3d_pred.shape[2]
    bt = B * T
    pred = points3d_pred.reshape(bt, n_pred, 3)
    obs_t = points3d_obs.reshape(bt, n_obs, 3).transpose(0, 2, 1)

    p_norm = jnp.sum(pred * pred, axis=2, keepdims=True)
    o_norm = jnp.sum(obs_t * obs_t, axis=1, keepdims=True)
    lhs = jnp.concatenate([pred, p_norm], axis=2)
    rhs = jnp.concatenate(
        [-2.0 * obs_t, jnp.ones((bt, 1, n_obs), jnp.float32)], axis=1)
    lhs_hi, lhs_lo = _split_bf16(lhs)
    rhs_hi, rhs_lo = _split_bf16(rhs)
    lhs_cat = jnp.concatenate([lhs_hi, lhs_hi, lhs_lo, lhs_lo], axis=2)
    rhs_cat = jnp.concatenate([rhs_hi, rhs_lo, rhs_hi, rhs_lo], axis=1)

    res = pl.pallas_call(
        functools.partial(_chamfer_body, n_pred=n_pred, n_obs=n_obs,
                          chunk=512),
        grid=(bt,),
        in_specs=[
            pl.BlockSpec((1, n_pred, 16), lambda f: (f, 0, 0)),
            pl.BlockSpec((1, 16, n_obs), lambda f: (f, 0, 0)),
            pl.BlockSpec((1, 1, n_obs), lambda f: (f, 0, 0)),
        ],
        out_specs=pl.BlockSpec((1, 1, n_obs), lambda f: (f, 0, 0)),
        out_shape=jax.ShapeDtypeStruct((bt, 1, n_obs), jnp.float32),
    )(lhs_cat, rhs_cat, o_norm)

    n_row = T * n_obs
    res3 = res.reshape(B, n_row // 128, 128)
    loss = pl.pallas_call(
        functools.partial(_loss_body, n_batch=B, n_row=n_row),
        in_specs=[pl.BlockSpec((B, n_row // 128, 128), lambda: (0, 0, 0))],
        out_specs=pl.BlockSpec((1, 1), lambda: (0, 0)),
        out_shape=jax.ShapeDtypeStruct((1, 1), jnp.float32),
    )(res3)
    return loss[0, 0]
